# baseline (speedup 1.0000x reference)
import jax
import jax.numpy as jnp
from jax import lax
from jax.experimental import pallas as pl
from jax.experimental.pallas import tpu as pltpu
from jax.experimental.pallas import tpu_sc as plsc

B, A, F, D = 64, 512, 128, 16
LANES = 16
W = F // 2
NGROUPS = W // LANES

NC, NS = 2, 16
NW = NC * NS
MOLS_PER_W = B // NW
ACHUNK = 128
NACH = A // ACHUNK


def _dyn_gather(vec, idx):
    dn = lax.GatherDimensionNumbers(
        offset_dims=(), collapsed_slice_dims=(0,), start_index_map=(0,))
    return lax.gather(vec, idx[:, None], dn, (1,),
                      mode=lax.GatherScatterMode.PROMISE_IN_BOUNDS)


def _graph_pool_body(atoms_hbm, edges_hbm, out_hbm, atoms_v, edges_v, out_v, sem):
    wid = lax.axis_index("s") * NC + lax.axis_index("c")

    lanes = lax.broadcasted_iota(jnp.int32, (LANES,), 0)
    gbases = [lanes + g * LANES for g in range(NGROUPS)]
    evenidx = [2 * lanes + g * 2 * LANES for g in range(NGROUPS)]
    oddidx = [2 * lanes + g * 2 * LANES + 1 for g in range(NGROUPS)]
    dconsts = [jnp.full((LANES,), d, jnp.int32) for d in range(D)]

    for m in range(MOLS_PER_W):
        b = wid * MOLS_PER_W + m
        pltpu.sync_copy(atoms_hbm.at[b], atoms_v)
        pltpu.sync_copy(edges_hbm.at[b], edges_v)

        for ch in range(NACH):
            def atom_body(a, ch=ch):
                aa = ch * ACHUNK + a
                selfv = jnp.full((LANES,), aa, jnp.int32)
                av = jnp.full((LANES,), a, jnp.int32)
                ev = plsc.load_gather(edges_v, [selfv, lanes])
                accs = [
                    plsc.bitcast(
                        plsc.load_gather(atoms_v, [selfv, gbases[g]]),
                        jnp.bfloat16)
                    for g in range(NGROUPS)]
                for d in range(D):
                    rowv = _dyn_gather(ev, dconsts[d])
                    for g in range(NGROUPS):
                        w = plsc.load_gather(atoms_v, [rowv, gbases[g]])
                        accs[g] = jnp.maximum(
                            accs[g], plsc.bitcast(w, jnp.bfloat16))
                for g in range(NGROUPS):
                    evens, odds = plsc.unpack(
                        accs[g], format=plsc.PackFormat.INTERLEAVED)
                    plsc.store_scatter(out_v, [av, evenidx[g]], evens)
                    plsc.store_scatter(out_v, [av, oddidx[g]], odds)

            plsc.parallel_loop(0, ACHUNK)(atom_body)
            pltpu.sync_copy(out_v, out_hbm.at[b, pl.ds(ch * ACHUNK, ACHUNK)])


_graph_pool = pl.kernel(
    _graph_pool_body,
    out_type=jax.ShapeDtypeStruct((B, A, F), jnp.float32),
    mesh=plsc.VectorSubcoreMesh(core_axis_name="c", subcore_axis_name="s"),
    scratch_types=[
        pltpu.VMEM((A, W), jnp.int32),
        pltpu.VMEM((A, D), jnp.int32),
        pltpu.VMEM((ACHUNK, F), jnp.float32),
        pltpu.SemaphoreType.DMA,
    ],
    compiler_params=pltpu.CompilerParams(
        use_tc_tiling_on_sc=False, needs_layout_passes=False),
)


def kernel(atoms, edges):
    bits = jax.lax.bitcast_convert_type(atoms, jnp.int32)
    r = (bits + 0x7FFF + ((bits >> 16) & 1)) >> 16
    evens, odds = r[..., 0::2], r[..., 1::2]
    packed = (odds << 16) | (evens & 0xFFFF)
    return _graph_pool(packed, edges.astype(jnp.int32))


# speedup vs baseline: 8.1513x; 8.1513x over previous
import jax
import jax.numpy as jnp
from jax import lax
from jax.experimental import pallas as pl
from jax.experimental.pallas import tpu as pltpu
from jax.experimental.pallas import tpu_sc as plsc

B, A, F, D = 64, 512, 128, 16
LANES = 16
W = F // 2
WPAD = W + 1
NGROUPS = W // LANES

NC, NS = 2, 16
NW = NC * NS
MOLS_PER_W = B // NW
ACHUNK = 128
NACH = A // ACHUNK


def _dyn_gather(vec, idx):
    dn = lax.GatherDimensionNumbers(
        offset_dims=(), collapsed_slice_dims=(0,), start_index_map=(0,))
    return lax.gather(vec, idx[:, None], dn, (1,),
                      mode=lax.GatherScatterMode.PROMISE_IN_BOUNDS)


def _graph_pool_body(atoms_hbm, edges_hbm, out_hbm,
                     stage_v, atoms_v, edges_v, out_v, sem):
    wid = lax.axis_index("s") * NC + lax.axis_index("c")

    lanes = lax.broadcasted_iota(jnp.int32, (LANES,), 0)
    gbases = [lanes + g * LANES for g in range(NGROUPS)]
    evenidx = [2 * lanes + g * 2 * LANES for g in range(NGROUPS)]
    oddidx = [2 * lanes + g * 2 * LANES + 1 for g in range(NGROUPS)]
    dconsts = [jnp.full((LANES,), d, jnp.int32) for d in range(D)]

    for m in range(MOLS_PER_W):
        b = wid * MOLS_PER_W + m
        pltpu.sync_copy(edges_hbm.at[b], edges_v)

        for ch in range(NACH):
            pltpu.sync_copy(atoms_hbm.at[b, pl.ds(ch * ACHUNK, ACHUNK)], stage_v)

            def pack_row(r, ch=ch):
                rv = jnp.full((LANES,), r, jnp.int32)
                prv = jnp.full((LANES,), ch * ACHUNK + r, jnp.int32)
                for g in range(NGROUPS):
                    a = plsc.load_gather(stage_v, [rv, evenidx[g]])
                    o = plsc.load_gather(stage_v, [rv, oddidx[g]])
                    w = plsc.bitcast(
                        plsc.pack(a, o, format=plsc.PackFormat.INTERLEAVED),
                        jnp.int32)
                    plsc.store_scatter(atoms_v, [prv, gbases[g]], w)

            plsc.parallel_loop(0, ACHUNK)(pack_row)

        for ch in range(NACH):
            def atom_body(a, ch=ch):
                aa = ch * ACHUNK + a
                selfv = jnp.full((LANES,), aa, jnp.int32)
                av = jnp.full((LANES,), a, jnp.int32)
                ev = plsc.load_gather(edges_v, [selfv, lanes])
                accs = [
                    plsc.bitcast(
                        plsc.load_gather(atoms_v, [selfv, gbases[g]]),
                        jnp.bfloat16)
                    for g in range(NGROUPS)]
                for d in range(D):
                    rowv = _dyn_gather(ev, dconsts[d])
                    for g in range(NGROUPS):
                        w = plsc.load_gather(atoms_v, [rowv, gbases[g]])
                        accs[g] = jnp.maximum(
                            accs[g], plsc.bitcast(w, jnp.bfloat16))
                for g in range(NGROUPS):
                    evens, odds = plsc.unpack(
                        accs[g], format=plsc.PackFormat.INTERLEAVED)
                    plsc.store_scatter(out_v, [av, evenidx[g]], evens)
                    plsc.store_scatter(out_v, [av, oddidx[g]], odds)

            plsc.parallel_loop(0, ACHUNK)(atom_body)
            pltpu.sync_copy(out_v, out_hbm.at[b, pl.ds(ch * ACHUNK, ACHUNK)])


_graph_pool = pl.kernel(
    _graph_pool_body,
    out_type=jax.ShapeDtypeStruct((B, A, F), jnp.float32),
    mesh=plsc.VectorSubcoreMesh(core_axis_name="c", subcore_axis_name="s"),
    scratch_types=[
        pltpu.VMEM((ACHUNK, F), jnp.float32),
        pltpu.VMEM((A, WPAD), jnp.int32),
        pltpu.VMEM((A, D), jnp.int32),
        pltpu.VMEM((ACHUNK, F), jnp.float32),
        pltpu.SemaphoreType.DMA,
    ],
    compiler_params=pltpu.CompilerParams(
        use_tc_tiling_on_sc=False, needs_layout_passes=False),
)


def kernel(atoms, edges):
    return _graph_pool(atoms, edges.astype(jnp.int32))


# speedup vs baseline: 8.1656x; 1.0018x over previous
import jax
import jax.numpy as jnp
from jax import lax
from jax.experimental import pallas as pl
from jax.experimental.pallas import tpu as pltpu
from jax.experimental.pallas import tpu_sc as plsc

B, A, F, D = 64, 512, 128, 16
LANES = 16
W = F // 2
WPAD = W + 1
NGROUPS = W // LANES

NC, NS = 2, 16
NW = NC * NS
MOLS_PER_W = B // NW
ACHUNK = 128
NACH = A // ACHUNK


def _dyn_gather(vec, idx):
    dn = lax.GatherDimensionNumbers(
        offset_dims=(), collapsed_slice_dims=(0,), start_index_map=(0,))
    return lax.gather(vec, idx[:, None], dn, (1,),
                      mode=lax.GatherScatterMode.PROMISE_IN_BOUNDS)


def _graph_pool_body(atoms_hbm, edges_hbm, out_hbm,
                     stage_v, atoms_v, edges_v, out_v, sem):
    wid = lax.axis_index("s") * NC + lax.axis_index("c")

    lanes = lax.broadcasted_iota(jnp.int32, (LANES,), 0)
    gbases = [lanes + g * LANES for g in range(NGROUPS)]
    evenidx = [2 * lanes + g * 2 * LANES for g in range(NGROUPS)]
    oddidx = [2 * lanes + g * 2 * LANES + 1 for g in range(NGROUPS)]
    dconsts = [jnp.full((LANES,), d, jnp.int32) for d in range(D)]

    for m in range(MOLS_PER_W):
        b = wid * MOLS_PER_W + m
        pltpu.sync_copy(edges_hbm.at[pl.ds(b * A, A)], edges_v)

        for ch in range(NACH):
            pltpu.sync_copy(
                atoms_hbm.at[pl.ds(b * A + ch * ACHUNK, ACHUNK)], stage_v)

            def pack_row(r, ch=ch):
                rv = jnp.full((LANES,), r, jnp.int32)
                prv = jnp.full((LANES,), ch * ACHUNK + r, jnp.int32)
                for g in range(NGROUPS):
                    a = plsc.load_gather(stage_v, [rv, evenidx[g]])
                    o = plsc.load_gather(stage_v, [rv, oddidx[g]])
                    w = plsc.bitcast(
                        plsc.pack(a, o, format=plsc.PackFormat.INTERLEAVED),
                        jnp.int32)
                    plsc.store_scatter(atoms_v, [prv, gbases[g]], w)

            plsc.parallel_loop(0, ACHUNK)(pack_row)

        for ch in range(NACH):
            def atom_body(a, ch=ch):
                aa = ch * ACHUNK + a
                selfv = jnp.full((LANES,), aa, jnp.int32)
                av = jnp.full((LANES,), a, jnp.int32)
                ev = plsc.load_gather(edges_v, [selfv, lanes])
                accs = [
                    plsc.bitcast(
                        plsc.load_gather(atoms_v, [selfv, gbases[g]]),
                        jnp.bfloat16)
                    for g in range(NGROUPS)]
                for d in range(D):
                    rowv = _dyn_gather(ev, dconsts[d])
                    for g in range(NGROUPS):
                        w = plsc.load_gather(atoms_v, [rowv, gbases[g]])
                        accs[g] = jnp.maximum(
                            accs[g], plsc.bitcast(w, jnp.bfloat16))
                for g in range(NGROUPS):
                    evens, odds = plsc.unpack(
                        accs[g], format=plsc.PackFormat.INTERLEAVED)
                    plsc.store_scatter(out_v, [av, evenidx[g]], evens)
                    plsc.store_scatter(out_v, [av, oddidx[g]], odds)

            plsc.parallel_loop(0, ACHUNK)(atom_body)
            pltpu.sync_copy(
                out_v, out_hbm.at[pl.ds(b * A + ch * ACHUNK, ACHUNK)])


_graph_pool = pl.kernel(
    _graph_pool_body,
    out_type=jax.ShapeDtypeStruct((B * A, F), jnp.float32),
    mesh=plsc.VectorSubcoreMesh(core_axis_name="c", subcore_axis_name="s"),
    scratch_types=[
        pltpu.VMEM((ACHUNK, F), jnp.float32),
        pltpu.VMEM((A, WPAD), jnp.int32),
        pltpu.VMEM((A, D), jnp.int32),
        pltpu.VMEM((ACHUNK, F), jnp.float32),
        pltpu.SemaphoreType.DMA,
    ],
    compiler_params=pltpu.CompilerParams(
        use_tc_tiling_on_sc=False, needs_layout_passes=False),
)


def kernel(atoms, edges):
    out = _graph_pool(atoms.reshape(B * A, F),
                      edges.astype(jnp.int32).reshape(B * A, D))
    return out.reshape(B, A, F)


# speedup vs baseline: 8.1885x; 1.0028x over previous
import jax
import jax.numpy as jnp
from jax import lax
from jax.experimental import pallas as pl
from jax.experimental.pallas import tpu as pltpu
from jax.experimental.pallas import tpu_sc as plsc

B, A, F, D = 64, 512, 128, 16
LANES = 16
W = F // 2
WPAD = W + 1
NGROUPS = W // LANES

NC, NS = 2, 16
NW = NC * NS
MOLS_PER_W = B // NW
ACHUNK = 128
NACH = A // ACHUNK


def _dyn_gather(vec, idx):
    dn = lax.GatherDimensionNumbers(
        offset_dims=(), collapsed_slice_dims=(0,), start_index_map=(0,))
    return lax.gather(vec, idx[:, None], dn, (1,),
                      mode=lax.GatherScatterMode.PROMISE_IN_BOUNDS)


def _graph_pool_body(atoms_hbm, edges_hbm, out_hbm,
                     stage_v, atoms_v, edges_v, out_v, sem):
    wid = lax.axis_index("s") * NC + lax.axis_index("c")

    lanes = lax.broadcasted_iota(jnp.int32, (LANES,), 0)
    gbases = [lanes + g * LANES for g in range(NGROUPS)]
    evenidx = [2 * lanes + g * 2 * LANES for g in range(NGROUPS)]
    oddidx = [2 * lanes + g * 2 * LANES + 1 for g in range(NGROUPS)]
    zerov = jnp.zeros((LANES,), jnp.int32)
    rotv = (lanes + 1) & (LANES - 1)

    for m in range(MOLS_PER_W):
        b = wid * MOLS_PER_W + m
        pltpu.sync_copy(edges_hbm.at[pl.ds(b * A, A)], edges_v)

        for ch in range(NACH):
            pltpu.sync_copy(
                atoms_hbm.at[pl.ds(b * A + ch * ACHUNK, ACHUNK)], stage_v)

            def pack_row(r, ch=ch):
                rv = jnp.full((LANES,), r, jnp.int32)
                prv = jnp.full((LANES,), ch * ACHUNK + r, jnp.int32)
                for g in range(NGROUPS):
                    a = plsc.load_gather(stage_v, [rv, evenidx[g]])
                    o = plsc.load_gather(stage_v, [rv, oddidx[g]])
                    w = plsc.bitcast(
                        plsc.pack(a, o, format=plsc.PackFormat.INTERLEAVED),
                        jnp.int32)
                    plsc.store_scatter(atoms_v, [prv, gbases[g]], w)

            plsc.parallel_loop(0, ACHUNK)(pack_row)

        for ch in range(NACH):
            def atom_body(a, ch=ch):
                aa = ch * ACHUNK + a
                selfv = jnp.full((LANES,), aa, jnp.int32)
                av = jnp.full((LANES,), a, jnp.int32)
                ev = plsc.load_gather(edges_v, [selfv, lanes])
                accs = [
                    plsc.bitcast(
                        plsc.load_gather(atoms_v, [selfv, gbases[g]]),
                        jnp.bfloat16)
                    for g in range(NGROUPS)]
                for d in range(D):
                    rowv = _dyn_gather(ev, zerov)
                    if d + 1 < D:
                        ev = _dyn_gather(ev, rotv)
                    for g in range(NGROUPS):
                        w = plsc.load_gather(atoms_v, [rowv, gbases[g]])
                        accs[g] = jnp.maximum(
                            accs[g], plsc.bitcast(w, jnp.bfloat16))
                for g in range(NGROUPS):
                    evens, odds = plsc.unpack(
                        accs[g], format=plsc.PackFormat.INTERLEAVED)
                    plsc.store_scatter(out_v, [av, evenidx[g]], evens)
                    plsc.store_scatter(out_v, [av, oddidx[g]], odds)

            plsc.parallel_loop(0, ACHUNK)(atom_body)
            pltpu.sync_copy(
                out_v, out_hbm.at[pl.ds(b * A + ch * ACHUNK, ACHUNK)])


_graph_pool = pl.kernel(
    _graph_pool_body,
    out_type=jax.ShapeDtypeStruct((B * A, F), jnp.float32),
    mesh=plsc.VectorSubcoreMesh(core_axis_name="c", subcore_axis_name="s"),
    scratch_types=[
        pltpu.VMEM((ACHUNK, F), jnp.float32),
        pltpu.VMEM((A, WPAD), jnp.int32),
        pltpu.VMEM((A, D), jnp.int32),
        pltpu.VMEM((ACHUNK, F), jnp.float32),
        pltpu.SemaphoreType.DMA,
    ],
    compiler_params=pltpu.CompilerParams(
        use_tc_tiling_on_sc=False, needs_layout_passes=False),
)


def kernel(atoms, edges):
    out = _graph_pool(atoms.reshape(B * A, F),
                      edges.astype(jnp.int32).reshape(B * A, D))
    return out.reshape(B, A, F)


# speedup vs baseline: 8.3032x; 1.0140x over previous
import jax
import jax.numpy as jnp
from jax import lax
from jax.experimental import pallas as pl
from jax.experimental.pallas import tpu as pltpu
from jax.experimental.pallas import tpu_sc as plsc

B, A, F, D = 64, 512, 128, 16
LANES = 16
W = F // 2
WPAD = W + 1
NGROUPS = W // LANES

NC, NS = 2, 16
NW = NC * NS
MOLS_PER_W = B // NW
ACHUNK = 128
NACH = A // ACHUNK


def _dyn_gather(vec, idx):
    dn = lax.GatherDimensionNumbers(
        offset_dims=(), collapsed_slice_dims=(0,), start_index_map=(0,))
    return lax.gather(vec, idx[:, None], dn, (1,),
                      mode=lax.GatherScatterMode.PROMISE_IN_BOUNDS)


def _graph_pool_body(atoms_hbm, edges_hbm, out_hbm,
                     stage_v, atoms_v, edges_v, out_v, sem):
    wid = lax.axis_index("s") * NC + lax.axis_index("c")

    lanes = lax.broadcasted_iota(jnp.int32, (LANES,), 0)
    gbases = [lanes + g * LANES for g in range(NGROUPS)]
    colA = [lanes + g * 2 * LANES for g in range(NGROUPS)]
    colB = [lanes + g * 2 * LANES + LANES for g in range(NGROUPS)]
    zerov = jnp.zeros((LANES,), jnp.int32)
    rotv = (lanes + 1) & (LANES - 1)

    for m in range(MOLS_PER_W):
        b = wid * MOLS_PER_W + m
        pltpu.sync_copy(edges_hbm.at[pl.ds(b * A, A)], edges_v)

        for ch in range(NACH):
            pltpu.sync_copy(
                atoms_hbm.at[pl.ds(b * A + ch * ACHUNK, ACHUNK)], stage_v)

            def pack_row(r, ch=ch):
                rv = jnp.full((LANES,), r, jnp.int32)
                prv = jnp.full((LANES,), ch * ACHUNK + r, jnp.int32)
                for g in range(NGROUPS):
                    a = plsc.load_gather(stage_v, [rv, colA[g]])
                    o = plsc.load_gather(stage_v, [rv, colB[g]])
                    w = plsc.bitcast(
                        plsc.pack(a, o, format=plsc.PackFormat.INTERLEAVED),
                        jnp.int32)
                    plsc.store_scatter(atoms_v, [prv, gbases[g]], w)

            plsc.parallel_loop(0, ACHUNK)(pack_row)

        for ch in range(NACH):
            def atom_body(a, ch=ch):
                aa = ch * ACHUNK + a
                selfv = jnp.full((LANES,), aa, jnp.int32)
                av = jnp.full((LANES,), a, jnp.int32)
                ev = plsc.load_gather(edges_v, [selfv, lanes])
                accs = [
                    plsc.bitcast(
                        plsc.load_gather(atoms_v, [selfv, gbases[g]]),
                        jnp.bfloat16)
                    for g in range(NGROUPS)]
                for d in range(D):
                    rowv = _dyn_gather(ev, zerov)
                    if d + 1 < D:
                        ev = _dyn_gather(ev, rotv)
                    for g in range(NGROUPS):
                        w = plsc.load_gather(atoms_v, [rowv, gbases[g]])
                        accs[g] = jnp.maximum(
                            accs[g], plsc.bitcast(w, jnp.bfloat16))
                for g in range(NGROUPS):
                    evens, odds = plsc.unpack(
                        accs[g], format=plsc.PackFormat.INTERLEAVED)
                    plsc.store_scatter(out_v, [av, colA[g]], evens)
                    plsc.store_scatter(out_v, [av, colB[g]], odds)

            plsc.parallel_loop(0, ACHUNK)(atom_body)
            pltpu.sync_copy(
                out_v, out_hbm.at[pl.ds(b * A + ch * ACHUNK, ACHUNK)])


_graph_pool = pl.kernel(
    _graph_pool_body,
    out_type=jax.ShapeDtypeStruct((B * A, F), jnp.float32),
    mesh=plsc.VectorSubcoreMesh(core_axis_name="c", subcore_axis_name="s"),
    scratch_types=[
        pltpu.VMEM((ACHUNK, F), jnp.float32),
        pltpu.VMEM((A, WPAD), jnp.int32),
        pltpu.VMEM((A, D), jnp.int32),
        pltpu.VMEM((ACHUNK, F), jnp.float32),
        pltpu.SemaphoreType.DMA,
    ],
    compiler_params=pltpu.CompilerParams(
        use_tc_tiling_on_sc=False, needs_layout_passes=False),
)


def kernel(atoms, edges):
    out = _graph_pool(atoms.reshape(B * A, F),
                      edges.astype(jnp.int32).reshape(B * A, D))
    return out.reshape(B, A, F)
